# baseline (speedup 1.0000x reference)
import functools

import jax
import jax.numpy as jnp
from jax import lax
from jax.experimental import pallas as pl
from jax.experimental.pallas import tpu as pltpu
from jax.experimental.pallas import tpu_sc as plsc

B = 16384
EMB = 64
NC, NS = 2, 16
NW = NC * NS
BPW = B // NW
CHUNK = 128
NCHUNK = BPW // CHUNK

@functools.cache
def _get_sc_gather():
    mesh = plsc.VectorSubcoreMesh(
        core_axis_name="c", subcore_axis_name="s", num_cores=NC, num_subcores=NS
    )

    @functools.partial(
        pl.kernel,
        mesh=mesh,
        out_type=jax.ShapeDtypeStruct((3 * B, EMB), jnp.float32),
        scratch_types=[
            pltpu.VMEM((3 * BPW,), jnp.int32),
            pltpu.VMEM((3 * BPW, EMB), jnp.float32),
            pltpu.SemaphoreType.DMA,
        ],
        compiler_params=pltpu.CompilerParams(use_tc_tiling_on_sc=False),
    )
    def _sc_gather(idx_hbm, emb_hbm, out_hbm, idx_v, rows_v, sem):
        wid = lax.axis_index("s") * NC + lax.axis_index("c")
        base = wid * BPW
        for l in range(3):
            pltpu.sync_copy(
                idx_hbm.at[pl.ds(l * B + base, BPW)],
                idx_v.at[pl.ds(l * BPW, BPW)],
            )
        copies = []
        for l in range(3):
            for j in range(NCHUNK):
                copies.append(
                    pltpu.async_copy(
                        emb_hbm.at[idx_v.at[pl.ds(l * BPW + j * CHUNK, CHUNK)]],
                        rows_v.at[pl.ds(l * BPW + j * CHUNK, CHUNK)],
                        sem,
                    )
                )
        for c in copies:
            c.wait()
        for l in range(3):
            pltpu.sync_copy(
                rows_v.at[pl.ds(l * BPW, BPW)],
                out_hbm.at[pl.ds(l * B + base, BPW)],
            )

    return _sc_gather


RBLK = 2048
NBLK = B // RBLK


def _mlp_body(g_ref, w1a_ref, w1b_ref, b1_ref, w2_ref, b2_ref, w3_ref, b3_ref,
              out_ref):
    d1 = g_ref[0]
    d2 = g_ref[1]
    cc = g_ref[2]
    avg = (d1 + d2) * 0.5
    h1 = jnp.dot(avg, w1a_ref[...], preferred_element_type=jnp.float32)
    h1 += jnp.dot(cc, w1b_ref[...], preferred_element_type=jnp.float32)
    h1 = jnp.maximum(h1 + b1_ref[...], 0.0)
    h2 = jnp.dot(h1, w2_ref[...], preferred_element_type=jnp.float32)
    h2 = jnp.maximum(h2 + b2_ref[...], 0.0)
    z = jnp.sum(h2 * w3_ref[...], axis=1) + b3_ref[0, 0]
    out_ref[...] = 1.0 / (1.0 + jnp.exp(-z))


_mlp = pl.pallas_call(
    _mlp_body,
    grid=(NBLK,),
    in_specs=[
        pl.BlockSpec((3, RBLK, EMB), lambda i: (0, i, 0)),
        pl.BlockSpec((EMB, 256), lambda i: (0, 0)),
        pl.BlockSpec((EMB, 256), lambda i: (0, 0)),
        pl.BlockSpec((1, 256), lambda i: (0, 0)),
        pl.BlockSpec((256, 128), lambda i: (0, 0)),
        pl.BlockSpec((1, 128), lambda i: (0, 0)),
        pl.BlockSpec((1, 128), lambda i: (0, 0)),
        pl.BlockSpec((1, 1), lambda i: (0, 0), memory_space=pltpu.SMEM),
    ],
    out_specs=pl.BlockSpec((RBLK,), lambda i: (i,)),
    out_shape=jax.ShapeDtypeStruct((B,), jnp.float32),
)


def kernel(drug_list1, drug_list2, cond_list, emb, W1, b1, W2, b2, W3, b3):
    idx = jnp.concatenate(
        [drug_list1, drug_list2, cond_list]
    ).astype(jnp.int32)
    g = _get_sc_gather()(idx, emb).reshape(3, B, EMB)
    return _mlp(
        g,
        W1[:EMB],
        W1[EMB:],
        b1.reshape(1, -1),
        W2,
        b2.reshape(1, -1),
        W3.reshape(1, -1),
        b3.reshape(1, 1),
    )
